# baseline (speedup 1.0000x reference)
import jax
import jax.numpy as jnp
from jax.experimental import pallas as pl
from jax.experimental.pallas import tpu as pltpu

D_IN = 300
H = 256
EMB = 512
B = 16
L = 512

T = 128
NC = L // T


def _body(lens_ref, xf_ref, xb_ref, wihf_ref, whhf_ref, bgif_ref, bhnf_ref,
          wihb_ref, whhb_ref, bgib_ref, bhnb_ref, lng_ref, lnb_ref, out_ref,
          hf_ref, hb_ref, gif_ref, gib_ref):
    c = pl.program_id(0)

    @pl.when(c == 0)
    def _init():
        hf_ref[...] = jnp.zeros_like(hf_ref)
        hb_ref[...] = jnp.zeros_like(hb_ref)

    gif_ref[...] = (
        jnp.dot(xf_ref[...], wihf_ref[...], preferred_element_type=jnp.float32)
        + bgif_ref[...]).astype(jnp.bfloat16)
    gib_ref[...] = (
        jnp.dot(xb_ref[...], wihb_ref[...], preferred_element_type=jnp.float32)
        + bgib_ref[...]).astype(jnp.bfloat16)

    lens = lens_ref[...]
    whh_f = whhf_ref[...]
    whh_b = whhb_ref[...]
    bhn_f = bhnf_ref[...]
    bhn_b = bhnb_ref[...]

    def gru_half(h, gi, gh, bhn):
        r = jax.nn.sigmoid(gi[:, :H] + gh[:, :H])
        z = jax.nn.sigmoid(gi[:, H:2 * H] + gh[:, H:2 * H])
        n = jnp.tanh(gi[:, 2 * H:] + r * (gh[:, 2 * H:] + bhn))
        return (1.0 - z) * n + z * h

    def step_f(j, hf):
        t = c * T + j
        gi_f = gif_ref[pl.ds(j * B, B), :].astype(jnp.float32)
        gh_f = jnp.dot(hf.astype(jnp.bfloat16), whh_f,
                       preferred_element_type=jnp.float32)
        hf_new = gru_half(hf, gi_f, gh_f, bhn_f)
        return jnp.where(t < lens, hf_new, hf)

    def step_b(j, hb):
        tb = (L - 1) - (c * T + j)
        gi_b = gib_ref[pl.ds((T - 1 - j) * B, B), :].astype(jnp.float32)
        gh_b = jnp.dot(hb.astype(jnp.bfloat16), whh_b,
                       preferred_element_type=jnp.float32)
        hb_new = gru_half(hb, gi_b, gh_b, bhn_b)
        return jnp.where(tb < lens, hb_new, hb)

    hf = jax.lax.fori_loop(0, T, step_f, hf_ref[...], unroll=16)
    hb = jax.lax.fori_loop(0, T, step_b, hb_ref[...], unroll=16)
    hf_ref[...] = hf
    hb_ref[...] = hb

    @pl.when(c == NC - 1)
    def _finish():
        h = jnp.concatenate([hf, hb], axis=1)
        mu = jnp.mean(h, axis=1, keepdims=True)
        d = h - mu
        var = jnp.mean(d * d, axis=1, keepdims=True)
        out_ref[...] = d * jax.lax.rsqrt(var + 1e-5) * lng_ref[...] + lnb_ref[...]


@jax.jit
def _encode(xT, lens, wihf, whhf, bgif, bhnf, wihb, whhb, bgib, bhnb, lng,
            lnb):
    full = lambda shape: pl.BlockSpec(shape, lambda c: (0, 0))
    grid_spec = pltpu.PrefetchScalarGridSpec(
        num_scalar_prefetch=0,
        grid=(NC,),
        in_specs=[
            full((B, H)),
            pl.BlockSpec((T * B, D_IN), lambda c: (c, 0)),
            pl.BlockSpec((T * B, D_IN), lambda c: (NC - 1 - c, 0)),
            full((D_IN, 3 * H)),
            full((H, 3 * H)),
            full((1, 3 * H)),
            full((1, H)),
            full((D_IN, 3 * H)),
            full((H, 3 * H)),
            full((1, 3 * H)),
            full((1, H)),
            full((1, EMB)),
            full((1, EMB)),
        ],
        out_specs=pl.BlockSpec((B, EMB), lambda c: (0, 0)),
        scratch_shapes=[
            pltpu.VMEM((B, H), jnp.float32),
            pltpu.VMEM((B, H), jnp.float32),
            pltpu.VMEM((T * B, 3 * H), jnp.bfloat16),
            pltpu.VMEM((T * B, 3 * H), jnp.bfloat16),
        ],
    )
    return pl.pallas_call(
        _body,
        grid_spec=grid_spec,
        out_shape=jax.ShapeDtypeStruct((B, EMB), jnp.float32),
    )(lens, xT, xT, wihf, whhf, bgif, bhnf, wihb, whhb, bgib, bhnb, lng, lnb)


def kernel(words, W_ih_f, W_hh_f, b_ih_f, b_hh_f, W_ih_b, W_hh_b, b_ih_b,
           b_hh_b, ln_g, ln_b, lengths):
    lens = jnp.broadcast_to(
        jnp.maximum(lengths.astype(jnp.int32), 1).reshape(B, 1), (B, H))
    bf = jnp.bfloat16
    xT = jnp.transpose(words, (1, 0, 2)).reshape(L * B, D_IN).astype(bf)

    def fold(b_ih, b_hh):
        return (b_ih + jnp.concatenate(
            [b_hh[:2 * H], jnp.zeros((H,), jnp.float32)])).reshape(1, -1)

    return _encode(
        xT, lens,
        W_ih_f.T.astype(bf), W_hh_f.T.astype(bf),
        fold(b_ih_f, b_hh_f), b_hh_f[2 * H:].reshape(1, -1),
        W_ih_b.T.astype(bf), W_hh_b.T.astype(bf),
        fold(b_ih_b, b_hh_b), b_hh_b[2 * H:].reshape(1, -1),
        ln_g.reshape(1, -1), ln_b.reshape(1, -1))


# speedup vs baseline: 1.6684x; 1.6684x over previous
import jax
import jax.numpy as jnp
from jax.experimental import pallas as pl
from jax.experimental.pallas import tpu as pltpu

D_IN = 300
H = 256
EMB = 512
B = 16
L = 512

T = 128
NC = L // T


def _body(lens_ref, xf_ref, xb_ref, wihf_ref, whhf_ref, bgif_ref, bhnf_ref,
          wihb_ref, whhb_ref, bgib_ref, bhnb_ref, lng_ref, lnb_ref, out_ref,
          hf_ref, hb_ref, gif_ref, gib_ref):
    c = pl.program_id(0)

    @pl.when(c == 0)
    def _init():
        hf_ref[...] = jnp.zeros_like(hf_ref)
        hb_ref[...] = jnp.zeros_like(hb_ref)

    gif_ref[...] = (
        jnp.dot(xf_ref[...], wihf_ref[...], preferred_element_type=jnp.float32)
        + bgif_ref[...]).astype(jnp.bfloat16)
    gib_ref[...] = (
        jnp.dot(xb_ref[...], wihb_ref[...], preferred_element_type=jnp.float32)
        + bgib_ref[...]).astype(jnp.bfloat16)

    lens = lens_ref[...]
    whh_f = whhf_ref[...]
    whh_b = whhb_ref[...]
    bhn_f = bhnf_ref[...]
    bhn_b = bhnb_ref[...]

    def gru_half(h, gi, gh, bhn):
        r = jax.nn.sigmoid(gi[:, :H] + gh[:, :H])
        z = jax.nn.sigmoid(gi[:, H:2 * H] + gh[:, H:2 * H])
        n = jnp.tanh(gi[:, 2 * H:] + r * (gh[:, 2 * H:] + bhn))
        return (1.0 - z) * n + z * h

    def step(j, carry):
        hf, hb = carry
        t = c * T + j
        gi_f = gif_ref[pl.ds(j * B, B), :].astype(jnp.float32)
        gh_f = jnp.dot(hf.astype(jnp.bfloat16), whh_f,
                       preferred_element_type=jnp.float32)
        hf_new = gru_half(hf, gi_f, gh_f, bhn_f)
        hf = jnp.where(t < lens, hf_new, hf)

        tb = (L - 1) - t
        gi_b = gib_ref[pl.ds((T - 1 - j) * B, B), :].astype(jnp.float32)
        gh_b = jnp.dot(hb.astype(jnp.bfloat16), whh_b,
                       preferred_element_type=jnp.float32)
        hb_new = gru_half(hb, gi_b, gh_b, bhn_b)
        hb = jnp.where(tb < lens, hb_new, hb)
        return hf, hb

    hf, hb = jax.lax.fori_loop(0, T, step, (hf_ref[...], hb_ref[...]),
                               unroll=32)
    hf_ref[...] = hf
    hb_ref[...] = hb

    @pl.when(c == NC - 1)
    def _finish():
        h = jnp.concatenate([hf, hb], axis=1)
        mu = jnp.mean(h, axis=1, keepdims=True)
        d = h - mu
        var = jnp.mean(d * d, axis=1, keepdims=True)
        out_ref[...] = d * jax.lax.rsqrt(var + 1e-5) * lng_ref[...] + lnb_ref[...]


@jax.jit
def _encode(xT, lens, wihf, whhf, bgif, bhnf, wihb, whhb, bgib, bhnb, lng,
            lnb):
    full = lambda shape: pl.BlockSpec(shape, lambda c: (0, 0))
    grid_spec = pltpu.PrefetchScalarGridSpec(
        num_scalar_prefetch=0,
        grid=(NC,),
        in_specs=[
            full((B, H)),
            pl.BlockSpec((T * B, D_IN), lambda c: (c, 0)),
            pl.BlockSpec((T * B, D_IN), lambda c: (NC - 1 - c, 0)),
            full((D_IN, 3 * H)),
            full((H, 3 * H)),
            full((1, 3 * H)),
            full((1, H)),
            full((D_IN, 3 * H)),
            full((H, 3 * H)),
            full((1, 3 * H)),
            full((1, H)),
            full((1, EMB)),
            full((1, EMB)),
        ],
        out_specs=pl.BlockSpec((B, EMB), lambda c: (0, 0)),
        scratch_shapes=[
            pltpu.VMEM((B, H), jnp.float32),
            pltpu.VMEM((B, H), jnp.float32),
            pltpu.VMEM((T * B, 3 * H), jnp.bfloat16),
            pltpu.VMEM((T * B, 3 * H), jnp.bfloat16),
        ],
    )
    return pl.pallas_call(
        _body,
        grid_spec=grid_spec,
        out_shape=jax.ShapeDtypeStruct((B, EMB), jnp.float32),
    )(lens, xT, xT, wihf, whhf, bgif, bhnf, wihb, whhb, bgib, bhnb, lng, lnb)


def kernel(words, W_ih_f, W_hh_f, b_ih_f, b_hh_f, W_ih_b, W_hh_b, b_ih_b,
           b_hh_b, ln_g, ln_b, lengths):
    lens = jnp.broadcast_to(
        jnp.maximum(lengths.astype(jnp.int32), 1).reshape(B, 1), (B, H))
    bf = jnp.bfloat16
    xT = jnp.transpose(words, (1, 0, 2)).reshape(L * B, D_IN).astype(bf)

    def fold(b_ih, b_hh):
        return (b_ih + jnp.concatenate(
            [b_hh[:2 * H], jnp.zeros((H,), jnp.float32)])).reshape(1, -1)

    return _encode(
        xT, lens,
        W_ih_f.T.astype(bf), W_hh_f.T.astype(bf),
        fold(b_ih_f, b_hh_f), b_hh_f[2 * H:].reshape(1, -1),
        W_ih_b.T.astype(bf), W_hh_b.T.astype(bf),
        fold(b_ih_b, b_hh_b), b_hh_b[2 * H:].reshape(1, -1),
        ln_g.reshape(1, -1), ln_b.reshape(1, -1))
